# baseline (speedup 1.0000x reference)
import functools

import jax
import jax.numpy as jnp
from jax import lax
from jax.experimental import pallas as pl
from jax.experimental.pallas import tpu as pltpu
from jax.experimental.pallas import tpu_sc as plsc

_N_BINS = 15
_SC_LANES = 16
_SC_WORKERS = 32
_SC_CHUNK = 2048


def _conf_kernel(*refs):
    labels_ref, v_ref = refs[-2], refs[-1]
    lrefs = refs[:-2]
    sub = lrefs[0].shape[0]
    for d, lref in enumerate(lrefs):
        x = lref[...]
        b_rows, c = x.shape
        xt = x.T
        m = jnp.max(xt, axis=0, keepdims=True)
        s = jnp.sum(jnp.exp(xt - m), axis=0, keepdims=True)
        idr = lax.broadcasted_iota(jnp.int32, (c, b_rows), 0).astype(
            jnp.float32)
        am = jnp.min(jnp.where(xt == m, idr, jnp.float32(c)), axis=0,
                     keepdims=True)
        conf = 1.0 / s
        conf = jnp.where(conf == 1.0, jnp.float32(0.999999), conf)
        lab = labels_ref[0, 0, pl.ds(d * sub, sub)].reshape(1, sub)
        v = jnp.where(am == lab, conf, -conf)
        v_ref[0, 0, pl.ds(d * sub, sub)] = v.reshape(sub)


def _sc_hist_fn(v2, n_chunks, chunks_per_row):
    mesh = plsc.VectorSubcoreMesh(core_axis_name="c", subcore_axis_name="s")

    @functools.partial(
        pl.kernel,
        mesh=mesh,
        out_type=jax.ShapeDtypeStruct((_SC_WORKERS, _N_BINS, 64), jnp.float32),
        scratch_types=[pltpu.VMEM((_N_BINS, 64), jnp.float32)],
        compiler_params=pltpu.CompilerParams(needs_layout_passes=False),
    )
    def hist_kernel(v_hbm, out_hbm, tab):
        zeros = jnp.zeros((_SC_LANES,), jnp.float32)

        @pl.loop(0, _N_BINS)
        def _zero_rows(b):
            @pl.loop(0, 4)
            def _zero_planes(j):
                tab[b, pl.ds(j * _SC_LANES, _SC_LANES)] = zeros

        lane = lax.iota(jnp.int32, _SC_LANES)
        ones = jnp.ones((_SC_LANES,), jnp.float32)

        def body(in_vmem):
            @pl.loop(0, _SC_CHUNK, step=_SC_LANES)
            def _(cstart):
                x = in_vmem[0, 0, pl.ds(cstart, _SC_LANES)]
                a = jnp.abs(x)
                pos = x > 0.0
                b = jnp.minimum((a * jnp.float32(_N_BINS)).astype(jnp.int32),
                                _N_BINS - 1)
                jl = jnp.where(pos, _SC_LANES, 0) + lane
                plsc.addupdate_scatter(tab, [b, jl], ones)
                plsc.addupdate_scatter(tab, [b, jl + 2 * _SC_LANES], a)

        pltpu.emit_pipeline(
            body,
            grid=(n_chunks,),
            in_specs=[pl.BlockSpec(
                (1, 1, _SC_CHUNK),
                lambda i: (i // chunks_per_row, 0, i % chunks_per_row))],
            core_axis_name=("c", "s"),
            dimension_semantics=(pltpu.PARALLEL,),
        )(v_hbm)

        wid = lax.axis_index("s") * 2 + lax.axis_index("c")
        pltpu.sync_copy(tab, out_hbm.at[wid])

    return hist_kernel(v2)


def _combine_kernel(part_ref, out_ref, *, n_total):
    p = part_ref[...]
    t = jnp.sum(p, axis=0)
    plane = lax.broadcasted_iota(jnp.int32, (_N_BINS, 64), 1) // _SC_LANES
    zero = jnp.zeros_like(t)
    cnt = jnp.sum(jnp.where(plane <= 1, t, zero), axis=1)
    scorr = jnp.sum(jnp.where(plane == 1, t, zero), axis=1)
    sconf = jnp.sum(jnp.where(plane >= 2, t, zero), axis=1)
    contrib = jnp.where(cnt > 20.0, jnp.abs(sconf - scorr), 0.0)
    out_ref[...] = (jnp.sum(contrib) / jnp.float32(n_total)).reshape(1, 1)


def kernel(logits, labels):
    n, c = logits.shape
    nsplit = 2
    sub = 16384
    block = nsplit * sub
    grid = n // block
    n_rows = n // _SC_CHUNK

    def _mk_spec(d):
        return pl.BlockSpec((sub, c), lambda i, d=d: (i * nsplit + d, 0))

    labelsf = labels.astype(jnp.float32).reshape(grid, 1, block)
    v2 = pl.pallas_call(
        _conf_kernel,
        grid=(grid,),
        in_specs=[_mk_spec(d) for d in range(nsplit)] + [
            pl.BlockSpec((1, 1, block), lambda i: (i, 0, 0)),
        ],
        out_specs=pl.BlockSpec((1, 1, block), lambda i: (i, 0, 0)),
        out_shape=jax.ShapeDtypeStruct((grid, 1, block), jnp.float32),
        compiler_params=pltpu.CompilerParams(
            dimension_semantics=("parallel",),
        ),
    )(*([logits] * nsplit), labelsf)

    partials = _sc_hist_fn(v2, n_rows, block // _SC_CHUNK)

    ece = pl.pallas_call(
        functools.partial(_combine_kernel, n_total=n),
        out_shape=jax.ShapeDtypeStruct((1, 1), jnp.float32),
    )(partials)
    return ece.reshape(1)


# speedup vs baseline: 1.0021x; 1.0021x over previous
import functools

import jax
import jax.numpy as jnp
from jax import lax
from jax.experimental import pallas as pl
from jax.experimental.pallas import tpu as pltpu
from jax.experimental.pallas import tpu_sc as plsc

_N_BINS = 15
_SC_LANES = 16
_SC_WORKERS = 32
_SC_CHUNK = 2048


def _conf_kernel(*refs):
    labels_ref, v_ref = refs[-2], refs[-1]
    lrefs = refs[:-2]
    sub = lrefs[0].shape[0]
    for d, lref in enumerate(lrefs):
        x = lref[...]
        b_rows, c = x.shape
        xt = x.T
        m = jnp.max(xt, axis=0, keepdims=True)
        s = jnp.sum(jnp.exp(xt - m), axis=0, keepdims=True)
        idr = lax.broadcasted_iota(jnp.int32, (c, b_rows), 0).astype(
            jnp.float32)
        am = jnp.min(jnp.where(xt == m, idr, jnp.float32(c)), axis=0,
                     keepdims=True)
        conf = 1.0 / s
        conf = jnp.where(conf == 1.0, jnp.float32(0.999999), conf)
        lab = labels_ref[0, 0, pl.ds(d * sub, sub)].reshape(1, sub)
        v = jnp.where(am == lab, conf, -conf)
        v_ref[0, 0, pl.ds(d * sub, sub)] = v.reshape(sub)


def _sc_hist_fn(v2, n_chunks, chunks_per_row):
    mesh = plsc.VectorSubcoreMesh(core_axis_name="c", subcore_axis_name="s")

    @functools.partial(
        pl.kernel,
        mesh=mesh,
        out_type=jax.ShapeDtypeStruct((_SC_WORKERS, _N_BINS, 64), jnp.float32),
        scratch_types=[pltpu.VMEM((_N_BINS, 64), jnp.float32)],
        compiler_params=pltpu.CompilerParams(needs_layout_passes=False),
    )
    def hist_kernel(v_hbm, out_hbm, tab):
        zeros = jnp.zeros((_SC_LANES,), jnp.float32)

        @pl.loop(0, _N_BINS)
        def _zero_rows(b):
            @pl.loop(0, 4)
            def _zero_planes(j):
                tab[b, pl.ds(j * _SC_LANES, _SC_LANES)] = zeros

        lane = lax.iota(jnp.int32, _SC_LANES)
        ones = jnp.ones((_SC_LANES,), jnp.float32)

        def body(in_vmem):
            @pl.loop(0, _SC_CHUNK, step=_SC_LANES)
            def _(cstart):
                x = in_vmem[0, 0, pl.ds(cstart, _SC_LANES)]
                a = jnp.abs(x)
                pos = x > 0.0
                b = jnp.minimum((a * jnp.float32(_N_BINS)).astype(jnp.int32),
                                _N_BINS - 1)
                jl = jnp.where(pos, _SC_LANES, 0) + lane
                plsc.addupdate_scatter(tab, [b, jl], ones)
                plsc.addupdate_scatter(tab, [b, jl + 2 * _SC_LANES], a)

        pltpu.emit_pipeline(
            body,
            grid=(n_chunks,),
            in_specs=[pl.BlockSpec(
                (1, 1, _SC_CHUNK),
                lambda i: (i // chunks_per_row, 0, i % chunks_per_row))],
            core_axis_name=("c", "s"),
            dimension_semantics=(pltpu.PARALLEL,),
        )(v_hbm)

        wid = lax.axis_index("s") * 2 + lax.axis_index("c")
        pltpu.sync_copy(tab, out_hbm.at[wid])

    return hist_kernel(v2)


def _combine_kernel(part_ref, out_ref, *, n_total):
    p = part_ref[...]
    t = jnp.sum(p, axis=0)
    plane = lax.broadcasted_iota(jnp.int32, (_N_BINS, 64), 1) // _SC_LANES
    zero = jnp.zeros_like(t)
    cnt = jnp.sum(jnp.where(plane <= 1, t, zero), axis=1)
    scorr = jnp.sum(jnp.where(plane == 1, t, zero), axis=1)
    sconf = jnp.sum(jnp.where(plane >= 2, t, zero), axis=1)
    contrib = jnp.where(cnt > 20.0, jnp.abs(sconf - scorr), 0.0)
    out_ref[...] = (jnp.sum(contrib) / jnp.float32(n_total)).reshape(1, 1)


def kernel(logits, labels):
    n, c = logits.shape
    nsplit = 4
    sub = 8192
    block = nsplit * sub
    grid = n // block
    n_rows = n // _SC_CHUNK

    def _mk_spec(d):
        return pl.BlockSpec((sub, c), lambda i, d=d: (i * nsplit + d, 0))

    labelsf = labels.astype(jnp.float32).reshape(grid, 1, block)
    v2 = pl.pallas_call(
        _conf_kernel,
        grid=(grid,),
        in_specs=[_mk_spec(d) for d in range(nsplit)] + [
            pl.BlockSpec((1, 1, block), lambda i: (i, 0, 0)),
        ],
        out_specs=pl.BlockSpec((1, 1, block), lambda i: (i, 0, 0)),
        out_shape=jax.ShapeDtypeStruct((grid, 1, block), jnp.float32),
        compiler_params=pltpu.CompilerParams(
            dimension_semantics=("parallel",),
        ),
    )(*([logits] * nsplit), labelsf)

    partials = _sc_hist_fn(v2, n_rows, block // _SC_CHUNK)

    ece = pl.pallas_call(
        functools.partial(_combine_kernel, n_total=n),
        out_shape=jax.ShapeDtypeStruct((1, 1), jnp.float32),
    )(partials)
    return ece.reshape(1)


# speedup vs baseline: 1.0030x; 1.0008x over previous
import functools

import jax
import jax.numpy as jnp
from jax import lax
from jax.experimental import pallas as pl
from jax.experimental.pallas import tpu as pltpu
from jax.experimental.pallas import tpu_sc as plsc

_N_BINS = 15
_SC_LANES = 16
_SC_WORKERS = 32
_SC_CHUNK = 2048


def _conf_kernel(*refs):
    labels_ref, v_ref = refs[-2], refs[-1]
    lrefs = refs[:-2]
    sub = lrefs[0].shape[0]
    for d, lref in enumerate(lrefs):
        x = lref[...]
        b_rows, c = x.shape
        xt = x.T
        m = jnp.max(xt, axis=0, keepdims=True)
        s = jnp.sum(jnp.exp(xt - m), axis=0, keepdims=True)
        idr = lax.broadcasted_iota(jnp.int32, (c, b_rows), 0).astype(
            jnp.float32)
        am = jnp.min(jnp.where(xt == m, idr, jnp.float32(c)), axis=0,
                     keepdims=True)
        conf = 1.0 / s
        conf = jnp.where(conf == 1.0, jnp.float32(0.999999), conf)
        lab = labels_ref[0, 0, pl.ds(d * sub, sub)].reshape(1, sub)
        v = jnp.where(am == lab, conf, -conf)
        v_ref[0, 0, pl.ds(d * sub, sub)] = v.reshape(sub)


def _sc_hist_fn(v2, n_chunks, chunks_per_row):
    mesh = plsc.VectorSubcoreMesh(core_axis_name="c", subcore_axis_name="s")

    @functools.partial(
        pl.kernel,
        mesh=mesh,
        out_type=jax.ShapeDtypeStruct((_SC_WORKERS, _N_BINS, 64), jnp.float32),
        scratch_types=[pltpu.VMEM((_N_BINS, 64), jnp.float32)],
        compiler_params=pltpu.CompilerParams(needs_layout_passes=False),
    )
    def hist_kernel(v_hbm, out_hbm, tab):
        zeros = jnp.zeros((_SC_LANES,), jnp.float32)

        @pl.loop(0, _N_BINS)
        def _zero_rows(b):
            @pl.loop(0, 4)
            def _zero_planes(j):
                tab[b, pl.ds(j * _SC_LANES, _SC_LANES)] = zeros

        lane = lax.iota(jnp.int32, _SC_LANES)
        ones = jnp.ones((_SC_LANES,), jnp.float32)

        def body(in_vmem):
            @pl.loop(0, _SC_CHUNK, step=4 * _SC_LANES)
            def _(cstart):
                for k in range(4):
                    x = in_vmem[0, 0, pl.ds(cstart + k * _SC_LANES,
                                            _SC_LANES)]
                    a = jnp.abs(x)
                    pos = x > 0.0
                    b = jnp.minimum(
                        (a * jnp.float32(_N_BINS)).astype(jnp.int32),
                        _N_BINS - 1)
                    jl = jnp.where(pos, _SC_LANES, 0) + lane
                    plsc.addupdate_scatter(tab, [b, jl], ones)
                    plsc.addupdate_scatter(tab, [b, jl + 2 * _SC_LANES], a)

        pltpu.emit_pipeline(
            body,
            grid=(n_chunks,),
            in_specs=[pl.BlockSpec(
                (1, 1, _SC_CHUNK),
                lambda i: (i // chunks_per_row, 0, i % chunks_per_row))],
            core_axis_name=("c", "s"),
            dimension_semantics=(pltpu.PARALLEL,),
        )(v_hbm)

        wid = lax.axis_index("s") * 2 + lax.axis_index("c")
        pltpu.sync_copy(tab, out_hbm.at[wid])

    return hist_kernel(v2)


def _combine_kernel(part_ref, out_ref, *, n_total):
    p = part_ref[...]
    t = jnp.sum(p, axis=0)
    plane = lax.broadcasted_iota(jnp.int32, (_N_BINS, 64), 1) // _SC_LANES
    zero = jnp.zeros_like(t)
    cnt = jnp.sum(jnp.where(plane <= 1, t, zero), axis=1)
    scorr = jnp.sum(jnp.where(plane == 1, t, zero), axis=1)
    sconf = jnp.sum(jnp.where(plane >= 2, t, zero), axis=1)
    contrib = jnp.where(cnt > 20.0, jnp.abs(sconf - scorr), 0.0)
    out_ref[...] = (jnp.sum(contrib) / jnp.float32(n_total)).reshape(1, 1)


def kernel(logits, labels):
    n, c = logits.shape
    nsplit = 4
    sub = 8192
    block = nsplit * sub
    grid = n // block
    n_rows = n // _SC_CHUNK

    def _mk_spec(d):
        return pl.BlockSpec((sub, c), lambda i, d=d: (i * nsplit + d, 0))

    labelsf = labels.astype(jnp.float32).reshape(grid, 1, block)
    v2 = pl.pallas_call(
        _conf_kernel,
        grid=(grid,),
        in_specs=[_mk_spec(d) for d in range(nsplit)] + [
            pl.BlockSpec((1, 1, block), lambda i: (i, 0, 0)),
        ],
        out_specs=pl.BlockSpec((1, 1, block), lambda i: (i, 0, 0)),
        out_shape=jax.ShapeDtypeStruct((grid, 1, block), jnp.float32),
        compiler_params=pltpu.CompilerParams(
            dimension_semantics=("parallel",),
        ),
    )(*([logits] * nsplit), labelsf)

    partials = _sc_hist_fn(v2, n_rows, block // _SC_CHUNK)

    ece = pl.pallas_call(
        functools.partial(_combine_kernel, n_total=n),
        out_shape=jax.ShapeDtypeStruct((1, 1), jnp.float32),
    )(partials)
    return ece.reshape(1)
